# Initial kernel scaffold; baseline (speedup 1.0000x reference)
import jax
import jax.numpy as jnp
from jax.experimental import pallas as pl


def kernel(x, norm1_g, norm1_b, Wq, bq, Wk, bk, Wv, bv, norm2_g, norm2_b, gate_W, gate_b, W1, b1, W2, b2):
    raise NotImplementedError("write your pallas kernel here")


# speedup vs baseline: 2.5907x; 2.5907x over previous
import functools
import math

import jax
import jax.numpy as jnp
from jax.experimental import pallas as pl
from jax.experimental.pallas import tpu as pltpu

HID = 768
HEADS = 12
HD = HID // HEADS
E = 8
FF = HID * 4
S = 2048
BT = 256
NB = S // BT
EPS = 1e-5
NEG = -1e30


def _ln(x, g, b):
    m = jnp.mean(x, axis=-1, keepdims=True)
    xc = x - m
    v = jnp.mean(xc * xc, axis=-1, keepdims=True)
    return xc * jax.lax.rsqrt(v + EPS) * g + b


def _pre_kernel(x_ref, g1_ref, b1_ref, wv_ref, bv_ref, g2_ref, b2_ref,
                gw_ref, gb_ref, x1_ref, xn2_ref, eid_ref):
    x = x_ref[...]
    xn1 = _ln(x, g1_ref[...], b1_ref[...])
    v = jnp.dot(xn1, wv_ref[...], preferred_element_type=jnp.float32) + bv_ref[...]
    x1 = x + jnp.tile(v, (1, HEADS))
    x1_ref[...] = x1
    xn2 = _ln(x1, g2_ref[...], b2_ref[...])
    xn2_ref[...] = xn2
    logits = jnp.dot(xn2, gw_ref[...], preferred_element_type=jnp.float32) + gb_ref[...]
    best = logits[:, 0:1]
    arg = jnp.zeros((S, 1), jnp.int32)
    for j in range(1, E):
        lj = logits[:, j:j + 1]
        upd = lj > best
        arg = jnp.where(upd, j, arg)
        best = jnp.where(upd, lj, best)
    eid_ref[...] = arg


def _gelu_exact(h):
    return 0.5 * h * (1.0 + jax.lax.erf(h * (1.0 / math.sqrt(2.0))))


NC = 4
FC = FF // NC


def _moe_kernel(xn2_ref, eid_ref, w1_ref, b1_ref, w2_ref, b2_ref, out_ref):
    e = pl.program_id(0)
    c = pl.program_id(1)
    w1 = w1_ref[0].astype(jnp.bfloat16)
    w2 = w2_ref[0].astype(jnp.bfloat16)
    b1 = b1_ref[0]
    b2 = b2_ref[0]
    first = c == 0

    def body(b, _):
        rows = pl.ds(pl.multiple_of(b * BT, BT), BT)
        xb = xn2_ref[rows, :].astype(jnp.bfloat16)
        h = jnp.dot(xb, w1, preferred_element_type=jnp.float32) + b1
        h = _gelu_exact(h)
        py = jnp.dot(h.astype(jnp.bfloat16), w2, preferred_element_type=jnp.float32)
        prev = out_ref[rows, :]
        base = jnp.where(first, jnp.broadcast_to(b2, prev.shape), prev)
        mask = eid_ref[rows, :] == e
        out_ref[rows, :] = jnp.where(mask, base + py, prev)
        return 0

    jax.lax.fori_loop(0, NB, body, 0)


@functools.partial(jax.jit, static_argnums=())
def kernel(x, norm1_g, norm1_b, Wq, bq, Wk, bk, Wv, bv, norm2_g, norm2_b,
           gate_W, gate_b, W1, b1, W2, b2):
    del Wq, bq, Wk, bk
    b_, s_, h_ = x.shape
    x2d = x.reshape(s_, h_)
    gw_pad = jnp.pad(gate_W, ((0, 0), (0, 128 - E)))
    gb_pad = jnp.pad(gate_b, (0, 128 - E), constant_values=NEG)

    x1, xn2, eid = pl.pallas_call(
        _pre_kernel,
        out_shape=(
            jax.ShapeDtypeStruct((S, HID), jnp.float32),
            jax.ShapeDtypeStruct((S, HID), jnp.float32),
            jax.ShapeDtypeStruct((S, 1), jnp.int32),
        ),
    )(x2d, norm1_g.reshape(1, HID), norm1_b.reshape(1, HID),
      Wv, bv.reshape(1, HD), norm2_g.reshape(1, HID), norm2_b.reshape(1, HID),
      gw_pad, gb_pad.reshape(1, 128))

    moe = pl.pallas_call(
        _moe_kernel,
        grid=(E, NC),
        in_specs=[
            pl.BlockSpec((S, HID), lambda e, c: (0, 0)),
            pl.BlockSpec((S, 1), lambda e, c: (0, 0)),
            pl.BlockSpec((1, HID, FC), lambda e, c: (e, 0, c)),
            pl.BlockSpec((1, 1, FC), lambda e, c: (e, 0, c)),
            pl.BlockSpec((1, FC, HID), lambda e, c: (e, c, 0)),
            pl.BlockSpec((1, 1, HID), lambda e, c: (e, 0, 0)),
        ],
        out_specs=pl.BlockSpec((S, HID), lambda e, c: (0, 0)),
        out_shape=jax.ShapeDtypeStruct((S, HID), jnp.float32),
        compiler_params=pltpu.CompilerParams(
            vmem_limit_bytes=60 * 1024 * 1024,
            dimension_semantics=("arbitrary", "arbitrary"),
        ),
    )(xn2, eid, W1, b1.reshape(E, 1, FF), W2, b2.reshape(E, 1, HID))

    return (x1 + moe).reshape(b_, s_, h_)


# speedup vs baseline: 5.3594x; 2.0687x over previous
import functools
import math

import jax
import jax.numpy as jnp
from jax import lax
from jax.experimental import pallas as pl
from jax.experimental.pallas import tpu as pltpu
from jax.experimental.pallas import tpu_sc as plsc

HID = 768
HEADS = 12
HD = HID // HEADS
E = 8
FF = HID * 4
S = 2048
EPS = 1e-5
NEG = -1e30
import numpy as _np
ATTN_SCALE = float(_np.float32(12.0) * _np.float32(jnp.bfloat16(1.0 / 12.0)))

BT = 128
NC = 4
FC = FF // NC

NT_SORT = 16
TPS = S // NT_SORT
NW = 32
TPW = S // NW


def _ln(x, g, b):
    m = jnp.mean(x, axis=-1, keepdims=True)
    xc = x - m
    v = jnp.mean(xc * xc, axis=-1, keepdims=True)
    return xc * jax.lax.rsqrt(v + EPS) * g + b


def _pre_kernel(x_ref, g1_ref, b1_ref, wv_ref, bv_ref, g2_ref, b2_ref,
                gw_ref, gb_ref, x1_ref, xn2_ref, eid_ref):
    x = x_ref[...]
    xn1 = _ln(x, g1_ref[...], b1_ref[...])
    v = jnp.dot(xn1, wv_ref[...], preferred_element_type=jnp.float32) + bv_ref[...]
    out1 = v.astype(jnp.bfloat16).astype(jnp.float32) * ATTN_SCALE
    x1 = x + jnp.tile(out1, (1, HEADS))
    x1_ref[...] = x1
    xn2 = _ln(x1, g2_ref[...], b2_ref[...])
    xn2_ref[...] = xn2
    logits = jnp.dot(xn2, gw_ref[...], preferred_element_type=jnp.float32) + gb_ref[...]
    best = logits[:, 0:1]
    arg = jnp.zeros((S, 1), jnp.int32)
    for j in range(1, E):
        lj = logits[:, j:j + 1]
        upd = lj > best
        arg = jnp.where(upd, j, arg)
        best = jnp.where(upd, lj, best)
    eid_ref[...] = arg


def _gelu_exact(h):
    return 0.5 * h * (1.0 + jax.lax.erf(h * (1.0 / math.sqrt(2.0))))


def _route_kernel(eid_hbm, pos_hbm, off_hbm, cnt_hbm, eidv, posv, cntv, offv, allv, sem):
    wid = lax.axis_index("s")
    base = wid * TPS
    pltpu.sync_copy(eid_hbm.at[pl.ds(base, TPS)], eidv)

    lanes = lax.iota(jnp.int32, 16)

    cnt = jnp.zeros((16,), jnp.int32)
    for k in range(TPS // 16):
        ev = eidv[pl.ds(k * 16, 16)]
        for e0 in range(E):
            m = ev == e0
            pc = plsc.all_reduce_population_count(m)
            cnt = cnt + jnp.where(lanes == e0, pc, 0)
    cntv[...] = cnt
    pltpu.sync_copy(cntv, cnt_hbm.at[wid])
    plsc.subcore_barrier()

    pltpu.sync_copy(cnt_hbm, allv)
    total = jnp.zeros((16,), jnp.int32)
    prior = jnp.zeros((16,), jnp.int32)
    for w in range(NT_SORT):
        cv = allv[w]
        total = total + cv
        before = jnp.where(jnp.int32(w) < wid, 1, 0)
        prior = prior + cv * before
    excl = plsc.cumsum(total) - total
    run = excl + prior

    @pl.when(wid == 0)
    def _():
        offv[...] = excl
        pltpu.sync_copy(offv, off_hbm)

    for k in range(TPS // 16):
        ev = eidv[pl.ds(k * 16, 16)]
        posk = jnp.zeros((16,), jnp.int32)
        for e0 in range(E):
            m = ev == e0
            mi = jnp.where(m, 1, 0)
            csum = plsc.cumsum(mi)
            start = jnp.sum(run * jnp.where(lanes == e0, 1, 0))
            posk = jnp.where(m, start + csum - 1, posk)
            run = run + jnp.where(lanes == e0, jnp.sum(mi), 0)
        posv[pl.ds(k * 16, 16)] = posk
    pltpu.sync_copy(posv, pos_hbm.at[pl.ds(base, TPS)])


def _route(eid):
    mesh = plsc.VectorSubcoreMesh(core_axis_name="c", subcore_axis_name="s",
                                  num_cores=1)
    return pl.kernel(
        _route_kernel,
        out_type=(
            jax.ShapeDtypeStruct((S,), jnp.int32),
            jax.ShapeDtypeStruct((16,), jnp.int32),
            jax.ShapeDtypeStruct((NT_SORT, 16), jnp.int32),
        ),
        mesh=mesh,
        scratch_types=[
            pltpu.VMEM((TPS,), jnp.int32),
            pltpu.VMEM((TPS,), jnp.int32),
            pltpu.VMEM((16,), jnp.int32),
            pltpu.VMEM((16,), jnp.int32),
            pltpu.VMEM((NT_SORT, 16), jnp.int32),
            pltpu.SemaphoreType.DMA,
        ],
        compiler_params=pltpu.CompilerParams(needs_layout_passes=False),
    )(eid)


def _scatter_kernel(xn2_hbm, pos_hbm, xs_hbm, idxv, rowsv, sem):
    wid = lax.axis_index("s") * 2 + lax.axis_index("c")
    base = wid * TPW
    pltpu.sync_copy(pos_hbm.at[pl.ds(base, TPW)], idxv)
    pltpu.sync_copy(xn2_hbm.at[pl.ds(base, TPW)], rowsv)
    pltpu.async_copy(rowsv, xs_hbm.at[idxv], sem).wait()


def _scatter(xn2, pos):
    mesh = plsc.VectorSubcoreMesh(core_axis_name="c", subcore_axis_name="s")
    return pl.kernel(
        _scatter_kernel,
        out_type=jax.ShapeDtypeStruct((S, HID), jnp.float32),
        mesh=mesh,
        scratch_types=[
            pltpu.VMEM((TPW,), jnp.int32),
            pltpu.VMEM((TPW, HID), jnp.float32),
            pltpu.SemaphoreType.DMA,
        ],
    )(xn2, pos)


def _unsort_kernel(ys_hbm, pos_hbm, out_hbm, idxv, rowsv, sem):
    wid = lax.axis_index("s") * 2 + lax.axis_index("c")
    base = wid * TPW
    pltpu.sync_copy(pos_hbm.at[pl.ds(base, TPW)], idxv)
    pltpu.async_copy(ys_hbm.at[idxv], rowsv, sem).wait()
    pltpu.sync_copy(rowsv, out_hbm.at[pl.ds(base, TPW)])


def _unsort(ys, pos):
    mesh = plsc.VectorSubcoreMesh(core_axis_name="c", subcore_axis_name="s")
    return pl.kernel(
        _unsort_kernel,
        out_type=jax.ShapeDtypeStruct((S, HID), jnp.float32),
        mesh=mesh,
        scratch_types=[
            pltpu.VMEM((TPW,), jnp.int32),
            pltpu.VMEM((TPW, HID), jnp.float32),
            pltpu.SemaphoreType.DMA,
        ],
    )(ys, pos)


def _moe_kernel(off_ref, xs_ref, w1_ref, b1_ref, w2_ref, b2_ref, ys_ref):
    e = pl.program_id(0)
    c = pl.program_id(1)
    off0 = off_ref[e]
    off1 = jnp.where(e + 1 < E, off_ref[e + 1], S)
    bstart = lax.div(off0, BT)
    bend = lax.div(off1 + (BT - 1), BT)
    w1 = w1_ref[0].astype(jnp.bfloat16)
    w2 = w2_ref[0].astype(jnp.bfloat16)
    b1 = b1_ref[0]
    b2 = b2_ref[0]
    first = c == 0

    def body(b, _):
        row0 = pl.multiple_of(b * BT, BT)
        rows = pl.ds(row0, BT)
        xb = xs_ref[rows, :].astype(jnp.bfloat16)
        h = jnp.dot(xb, w1, preferred_element_type=jnp.float32) + b1
        h = _gelu_exact(h)
        py = jnp.dot(h.astype(jnp.bfloat16), w2, preferred_element_type=jnp.float32)
        row_ids = row0 + lax.broadcasted_iota(jnp.int32, (BT, 1), 0)
        mask = (row_ids >= off0) & (row_ids < off1)
        prev = ys_ref[rows, :]
        base = jnp.where(first, jnp.broadcast_to(b2, prev.shape), prev)
        ys_ref[rows, :] = jnp.where(mask, base + py, prev)
        return 0

    lax.fori_loop(bstart, bend, body, 0)


def _moe(xs, off, W1, b1, W2, b2):
    grid_spec = pltpu.PrefetchScalarGridSpec(
        num_scalar_prefetch=1,
        grid=(E, NC),
        in_specs=[
            pl.BlockSpec((S, HID), lambda e, c, off: (0, 0)),
            pl.BlockSpec((1, HID, FC), lambda e, c, off: (e, 0, c)),
            pl.BlockSpec((1, 1, FC), lambda e, c, off: (e, 0, c)),
            pl.BlockSpec((1, FC, HID), lambda e, c, off: (e, c, 0)),
            pl.BlockSpec((1, 1, HID), lambda e, c, off: (e, 0, 0)),
        ],
        out_specs=pl.BlockSpec((S, HID), lambda e, c, off: (0, 0)),
    )
    return pl.pallas_call(
        _moe_kernel,
        grid_spec=grid_spec,
        out_shape=jax.ShapeDtypeStruct((S, HID), jnp.float32),
        compiler_params=pltpu.CompilerParams(
            vmem_limit_bytes=60 * 1024 * 1024,
            dimension_semantics=("arbitrary", "arbitrary"),
        ),
    )(off, xs, W1, b1.reshape(E, 1, FF), W2, b2.reshape(E, 1, HID))


def kernel(x, norm1_g, norm1_b, Wq, bq, Wk, bk, Wv, bv, norm2_g, norm2_b,
           gate_W, gate_b, W1, b1, W2, b2):
    del Wq, bq, Wk, bk
    b_, s_, h_ = x.shape
    x2d = x.reshape(s_, h_)
    gw_pad = jnp.pad(gate_W, ((0, 0), (0, 128 - E)))
    gb_pad = jnp.pad(gate_b, (0, 128 - E), constant_values=NEG)

    x1, xn2, eid = pl.pallas_call(
        _pre_kernel,
        out_shape=(
            jax.ShapeDtypeStruct((S, HID), jnp.float32),
            jax.ShapeDtypeStruct((S, HID), jnp.float32),
            jax.ShapeDtypeStruct((S, 1), jnp.int32),
        ),
    )(x2d, norm1_g.reshape(1, HID), norm1_b.reshape(1, HID),
      Wv, bv.reshape(1, HD), norm2_g.reshape(1, HID), norm2_b.reshape(1, HID),
      gw_pad, gb_pad.reshape(1, 128))

    pos, off, _ = _route(eid.reshape(S))
    xs = _scatter(xn2, pos)
    ys = _moe(xs, off, W1, b1, W2, b2)
    moe_out = _unsort(ys, pos)

    return (x1 + moe_out).reshape(b_, s_, h_)


# speedup vs baseline: 5.9884x; 1.1174x over previous
import functools
import math

import jax
import jax.numpy as jnp
from jax import lax
from jax.experimental import pallas as pl
from jax.experimental.pallas import tpu as pltpu
from jax.experimental.pallas import tpu_sc as plsc

HID = 768
HEADS = 12
HD = HID // HEADS
E = 8
FF = HID * 4
S = 2048
EPS = 1e-5
NEG = -1e30
import numpy as _np
ATTN_SCALE = float(_np.float32(12.0) * _np.float32(jnp.bfloat16(1.0 / 12.0)))

BT = 128
NC = 2
FC = FF // NC

NT_SORT = 16
TPS = S // NT_SORT
NW = 32
TPW = S // NW


def _ln(x, g, b):
    m = jnp.mean(x, axis=-1, keepdims=True)
    xc = x - m
    v = jnp.mean(xc * xc, axis=-1, keepdims=True)
    return xc * jax.lax.rsqrt(v + EPS) * g + b


def _pre_kernel(x_ref, g1_ref, b1_ref, wv_ref, bv_ref, g2_ref, b2_ref,
                gw_ref, gb_ref, x1_ref, xn2_ref, eid_ref):
    x = x_ref[...]
    xn1 = _ln(x, g1_ref[...], b1_ref[...])
    v = jnp.dot(xn1, wv_ref[...], preferred_element_type=jnp.float32) + bv_ref[...]
    out1 = v.astype(jnp.bfloat16).astype(jnp.float32) * ATTN_SCALE
    x1 = x + jnp.tile(out1, (1, HEADS))
    x1_ref[...] = x1
    xn2 = _ln(x1, g2_ref[...], b2_ref[...])
    xn2_ref[...] = xn2
    logits = jnp.dot(xn2, gw_ref[...], preferred_element_type=jnp.float32) + gb_ref[...]
    best = logits[:, 0:1]
    arg = jnp.zeros((S, 1), jnp.int32)
    for j in range(1, E):
        lj = logits[:, j:j + 1]
        upd = lj > best
        arg = jnp.where(upd, j, arg)
        best = jnp.where(upd, lj, best)
    eid_ref[...] = arg


def _gelu_exact(h):
    return 0.5 * h * (1.0 + jax.lax.erf(h * (1.0 / math.sqrt(2.0))))


def _route_kernel(eid_hbm, pos_hbm, off_hbm, cnt_hbm, eidv, posv, cntv, offv, allv, sem):
    wid = lax.axis_index("s")
    base = wid * TPS
    pltpu.sync_copy(eid_hbm.at[pl.ds(base, TPS)], eidv)

    lanes = lax.iota(jnp.int32, 16)

    cnt = jnp.zeros((16,), jnp.int32)
    for k in range(TPS // 16):
        ev = eidv[pl.ds(k * 16, 16)]
        for e0 in range(E):
            m = ev == e0
            pc = plsc.all_reduce_population_count(m)
            cnt = cnt + jnp.where(lanes == e0, pc, 0)
    cntv[...] = cnt
    pltpu.sync_copy(cntv, cnt_hbm.at[wid])
    plsc.subcore_barrier()

    pltpu.sync_copy(cnt_hbm, allv)
    total = jnp.zeros((16,), jnp.int32)
    prior = jnp.zeros((16,), jnp.int32)
    for w in range(NT_SORT):
        cv = allv[w]
        total = total + cv
        before = jnp.where(jnp.int32(w) < wid, 1, 0)
        prior = prior + cv * before
    excl = plsc.cumsum(total) - total
    run = excl + prior

    @pl.when(wid == 0)
    def _():
        offv[...] = excl
        pltpu.sync_copy(offv, off_hbm)

    for k in range(TPS // 16):
        ev = eidv[pl.ds(k * 16, 16)]
        posk = jnp.zeros((16,), jnp.int32)
        for e0 in range(E):
            m = ev == e0
            mi = jnp.where(m, 1, 0)
            csum = plsc.cumsum(mi)
            start = jnp.sum(run * jnp.where(lanes == e0, 1, 0))
            posk = jnp.where(m, start + csum - 1, posk)
            run = run + jnp.where(lanes == e0, jnp.sum(mi), 0)
        posv[pl.ds(k * 16, 16)] = posk
    pltpu.sync_copy(posv, pos_hbm.at[pl.ds(base, TPS)])


def _route(eid):
    mesh = plsc.VectorSubcoreMesh(core_axis_name="c", subcore_axis_name="s",
                                  num_cores=1)
    return pl.kernel(
        _route_kernel,
        out_type=(
            jax.ShapeDtypeStruct((S,), jnp.int32),
            jax.ShapeDtypeStruct((16,), jnp.int32),
            jax.ShapeDtypeStruct((NT_SORT, 16), jnp.int32),
        ),
        mesh=mesh,
        scratch_types=[
            pltpu.VMEM((TPS,), jnp.int32),
            pltpu.VMEM((TPS,), jnp.int32),
            pltpu.VMEM((16,), jnp.int32),
            pltpu.VMEM((16,), jnp.int32),
            pltpu.VMEM((NT_SORT, 16), jnp.int32),
            pltpu.SemaphoreType.DMA,
        ],
        compiler_params=pltpu.CompilerParams(needs_layout_passes=False),
    )(eid)


def _scatter_kernel(xn2_hbm, pos_hbm, xs_hbm, idxv, rowsv, sem):
    wid = lax.axis_index("s") * 2 + lax.axis_index("c")
    base = wid * TPW
    pltpu.sync_copy(pos_hbm.at[pl.ds(base, TPW)], idxv)
    pltpu.sync_copy(xn2_hbm.at[pl.ds(base, TPW)], rowsv)
    pltpu.async_copy(rowsv, xs_hbm.at[idxv], sem).wait()


def _scatter(xn2, pos):
    mesh = plsc.VectorSubcoreMesh(core_axis_name="c", subcore_axis_name="s")
    return pl.kernel(
        _scatter_kernel,
        out_type=jax.ShapeDtypeStruct((S, HID), jnp.float32),
        mesh=mesh,
        scratch_types=[
            pltpu.VMEM((TPW,), jnp.int32),
            pltpu.VMEM((TPW, HID), jnp.float32),
            pltpu.SemaphoreType.DMA,
        ],
    )(xn2, pos)


def _unsort_kernel(ys_hbm, pos_hbm, out_hbm, idxv, rowsv, sem):
    wid = lax.axis_index("s") * 2 + lax.axis_index("c")
    base = wid * TPW
    pltpu.sync_copy(pos_hbm.at[pl.ds(base, TPW)], idxv)
    pltpu.async_copy(ys_hbm.at[idxv], rowsv, sem).wait()
    pltpu.sync_copy(rowsv, out_hbm.at[pl.ds(base, TPW)])


def _unsort(ys, pos):
    mesh = plsc.VectorSubcoreMesh(core_axis_name="c", subcore_axis_name="s")
    return pl.kernel(
        _unsort_kernel,
        out_type=jax.ShapeDtypeStruct((S, HID), jnp.float32),
        mesh=mesh,
        scratch_types=[
            pltpu.VMEM((TPW,), jnp.int32),
            pltpu.VMEM((TPW, HID), jnp.float32),
            pltpu.SemaphoreType.DMA,
        ],
    )(ys, pos)


def _moe_kernel(off_ref, xs_ref, w1_ref, b1_ref, w2_ref, b2_ref, ys_ref):
    e = pl.program_id(0)
    c = pl.program_id(1)
    off0 = off_ref[e]
    off1 = jnp.where(e + 1 < E, off_ref[e + 1], S)
    bstart = lax.div(off0, BT)
    bend = lax.div(off1 + (BT - 1), BT)
    w1 = w1_ref[0].astype(jnp.bfloat16)
    w2 = w2_ref[0].astype(jnp.bfloat16)
    b1 = b1_ref[0]
    b2 = b2_ref[0]
    first = c == 0

    def body(b, _):
        row0 = pl.multiple_of(b * BT, BT)
        rows = pl.ds(row0, BT)
        xb = xs_ref[rows, :].astype(jnp.bfloat16)
        h = jnp.dot(xb, w1, preferred_element_type=jnp.float32) + b1
        h = _gelu_exact(h)
        py = jnp.dot(h.astype(jnp.bfloat16), w2, preferred_element_type=jnp.float32)
        row_ids = row0 + lax.broadcasted_iota(jnp.int32, (BT, 1), 0)
        mask = (row_ids >= off0) & (row_ids < off1)
        prev = ys_ref[rows, :]
        base = jnp.where(first, jnp.broadcast_to(b2, prev.shape), prev)
        ys_ref[rows, :] = jnp.where(mask, base + py, prev)
        return 0

    lax.fori_loop(bstart, bend, body, 0)


def _moe(xs, off, W1, b1, W2, b2):
    grid_spec = pltpu.PrefetchScalarGridSpec(
        num_scalar_prefetch=1,
        grid=(E, NC),
        in_specs=[
            pl.BlockSpec((S, HID), lambda e, c, off: (0, 0)),
            pl.BlockSpec((1, HID, FC), lambda e, c, off: (e, 0, c)),
            pl.BlockSpec((1, 1, FC), lambda e, c, off: (e, 0, c)),
            pl.BlockSpec((1, FC, HID), lambda e, c, off: (e, c, 0)),
            pl.BlockSpec((1, 1, HID), lambda e, c, off: (e, 0, 0)),
        ],
        out_specs=pl.BlockSpec((S, HID), lambda e, c, off: (0, 0)),
    )
    return pl.pallas_call(
        _moe_kernel,
        grid_spec=grid_spec,
        out_shape=jax.ShapeDtypeStruct((S, HID), jnp.float32),
        compiler_params=pltpu.CompilerParams(
            vmem_limit_bytes=60 * 1024 * 1024,
            dimension_semantics=("arbitrary", "arbitrary"),
        ),
    )(off, xs, W1, b1.reshape(E, 1, FF), W2, b2.reshape(E, 1, HID))


def kernel(x, norm1_g, norm1_b, Wq, bq, Wk, bk, Wv, bv, norm2_g, norm2_b,
           gate_W, gate_b, W1, b1, W2, b2):
    del Wq, bq, Wk, bk
    b_, s_, h_ = x.shape
    x2d = x.reshape(s_, h_)
    gw_pad = jnp.pad(gate_W, ((0, 0), (0, 128 - E)))
    gb_pad = jnp.pad(gate_b, (0, 128 - E), constant_values=NEG)

    x1, xn2, eid = pl.pallas_call(
        _pre_kernel,
        out_shape=(
            jax.ShapeDtypeStruct((S, HID), jnp.float32),
            jax.ShapeDtypeStruct((S, HID), jnp.float32),
            jax.ShapeDtypeStruct((S, 1), jnp.int32),
        ),
    )(x2d, norm1_g.reshape(1, HID), norm1_b.reshape(1, HID),
      Wv, bv.reshape(1, HD), norm2_g.reshape(1, HID), norm2_b.reshape(1, HID),
      gw_pad, gb_pad.reshape(1, 128))

    pos, off, _ = _route(eid.reshape(S))
    xs = _scatter(xn2, pos)
    ys = _moe(xs, off, W1, b1, W2, b2)
    moe_out = _unsort(ys, pos)

    return (x1 + moe_out).reshape(b_, s_, h_)
